# Initial kernel scaffold; baseline (speedup 1.0000x reference)
import jax
import jax.numpy as jnp
from jax.experimental import pallas as pl


def kernel(x):
    raise NotImplementedError("write your pallas kernel here")


# speedup vs baseline: 5.0243x; 5.0243x over previous
import jax
import jax.numpy as jnp
from jax.experimental import pallas as pl

_K = 16
_SHIFT = _K - 1


def _span_kernel(x_cur_ref, x_nxt_ref, out_ref, *, tl, L):
    i = pl.program_id(1)
    cur = x_cur_ref[0]
    nxt = x_nxt_ref[0]
    shifted = jnp.concatenate([cur[_SHIFT:, :], nxt[:_SHIFT, :]], axis=0)
    row = jax.lax.broadcasted_iota(jnp.int32, shifted.shape, 0)
    g = i * tl + row + _SHIFT
    shifted = jnp.where(g < L, shifted, 0.0)
    out_ref[0, :, 0, :] = cur
    out_ref[0, :, 1, :] = shifted


def kernel(x):
    B, L, D = x.shape
    TL = 512
    nb = L // TL

    grid = (B, nb)
    out = pl.pallas_call(
        lambda a, b, o: _span_kernel(a, b, o, tl=TL, L=L),
        grid=grid,
        in_specs=[
            pl.BlockSpec((1, TL, D), lambda b, i: (b, i, 0)),
            pl.BlockSpec((1, TL, D), lambda b, i: (b, jnp.minimum(i + 1, nb - 1), 0)),
        ],
        out_specs=pl.BlockSpec((1, TL, 2, D), lambda b, i: (b, i, 0, 0)),
        out_shape=jax.ShapeDtypeStruct((B, L, 2, D), x.dtype),
    )(x, x)
    return out


# speedup vs baseline: 6.0645x; 1.2070x over previous
import jax
import jax.numpy as jnp
from jax.experimental import pallas as pl

_K = 16
_SHIFT = _K - 1


def _span_kernel(x_cur_ref, x_nxt_ref, out_ref, *, tl, L):
    i = pl.program_id(1)
    cur = x_cur_ref[0]
    nxt = x_nxt_ref[0]
    shifted = jnp.concatenate([cur[_SHIFT:, :], nxt[:_SHIFT, :]], axis=0)
    row = jax.lax.broadcasted_iota(jnp.int32, shifted.shape, 0)
    g = i * tl + row + _SHIFT
    shifted = jnp.where(g < L, shifted, 0.0)
    out_ref[0, :, 0, :] = cur
    out_ref[0, :, 1, :] = shifted


def kernel(x):
    B, L, D = x.shape
    TL = 512
    nb = L // TL

    grid = (B, nb)
    out = pl.pallas_call(
        lambda a, b, o: _span_kernel(a, b, o, tl=TL, L=L),
        grid=grid,
        in_specs=[
            pl.BlockSpec((1, TL, D), lambda b, i: (b, i, 0)),
            pl.BlockSpec(
                (1, 16, D),
                lambda b, i: (b, jnp.minimum((i + 1) * (TL // 16), L // 16 - 1), 0),
            ),
        ],
        out_specs=pl.BlockSpec((1, TL, 2, D), lambda b, i: (b, i, 0, 0)),
        out_shape=jax.ShapeDtypeStruct((B, L, 2, D), x.dtype),
    )(x, x)
    return out
